# baseline (speedup 1.0000x reference)
import functools

import jax
import jax.numpy as jnp
from jax import lax
from jax.experimental import pallas as pl
from jax.experimental.pallas import tpu as pltpu
from jax.experimental.pallas import tpu_sc as plsc

_NEG = -1e30
_L = 16
_NT = 16
_BIG = 2 ** 30


def _shuf(v, idx):
    return v.at[idx].get(mode="promise_in_bounds")


def _csplat(c):
    return jnp.broadcast_to(jnp.int32(c), (_L,))


def _bfly_max(v, lane):
    for s in (1, 2, 4, 8):
        v = jnp.maximum(v, _shuf(v, lane ^ s))
    return v


def _bfly_min(v, lane):
    for s in (1, 2, 4, 8):
        v = jnp.minimum(v, _shuf(v, lane ^ s))
    return v


def _sc_body(score_h, dx_h, dy_h, dw_h, dh_h, ax1_h, ay1_h, ax2_h, ay2_h,
             out_h,
             sco_v, x1_v, y1_v, x2_v, y2_v, ar_v,
             sa_v, sb_v, sc2_v, sd_v,
             publoc_v, pubbuf_v, ibuf_v, outbuf_v, pub_sh,
             *, k, thr, hmax, wmax, per_tile):
    cid = lax.axis_index("c")
    sid = lax.axis_index("s")
    nchunk = per_tile // _L
    lane = lax.iota(jnp.int32, _L)

    @pl.when(cid == 0)
    def _main():
        base = sid * per_tile
        pltpu.sync_copy(score_h.at[pl.ds(base, per_tile)], sco_v)
        pltpu.sync_copy(dx_h.at[pl.ds(base, per_tile)], sa_v)
        pltpu.sync_copy(dy_h.at[pl.ds(base, per_tile)], sb_v)
        pltpu.sync_copy(ax1_h.at[pl.ds(base, per_tile)], sc2_v)
        pltpu.sync_copy(ay1_h.at[pl.ds(base, per_tile)], sd_v)
        pltpu.sync_copy(ax2_h.at[pl.ds(base, per_tile)], x1_v)
        pltpu.sync_copy(ay2_h.at[pl.ds(base, per_tile)], y1_v)
        for i in range(nchunk):
            ds = pl.ds(i * _L, _L)
            s = sco_v[ds]
            a1 = sc2_v[ds]
            b1 = sd_v[ds]
            w = x1_v[ds] - a1
            h = y1_v[ds] - b1
            cx = a1 + 0.5 * w
            cy = b1 + 0.5 * h
            cx = cx + (s * sa_v[ds]) * w
            cy = cy + (s * sb_v[ds]) * h
            x2_v[ds] = cx
            y2_v[ds] = cy
        pltpu.sync_copy(dw_h.at[pl.ds(base, per_tile)], sa_v)
        pltpu.sync_copy(dh_h.at[pl.ds(base, per_tile)], sb_v)
        for i in range(nchunk):
            ds = pl.ds(i * _L, _L)
            s = sco_v[ds]
            w = (x1_v[ds] - sc2_v[ds]) * jnp.exp(s * sa_v[ds])
            h = (y1_v[ds] - sd_v[ds]) * jnp.exp(s * sb_v[ds])
            cx = x2_v[ds]
            cy = y2_v[ds]
            nx1 = jnp.maximum(cx - 0.5 * w, 0.0)
            ny1 = jnp.maximum(cy - 0.5 * h, 0.0)
            nx2 = jnp.minimum(cx + 0.5 * w, wmax)
            ny2 = jnp.minimum(cy + 0.5 * h, hmax)
            x1_v[ds] = nx1
            y1_v[ds] = ny1
            x2_v[ds] = nx2
            y2_v[ds] = ny2
            ar_v[ds] = jnp.maximum(nx2 - nx1, 0.0) * jnp.maximum(ny2 - ny1, 0.0)

        lanef = lane.astype(jnp.float32)
        basef = base.astype(jnp.float32)
        bigf = jnp.float32(2.0 ** 30)

        def publish(m, mif):
            mm = _bfly_max(m, lane)
            gidxf = mif * float(_L) + lanef + basef
            lwinf = _bfly_min(jnp.where(m == mm, gidxf, bigf), lane)
            lofsi = (lwinf - basef).astype(jnp.int32)
            lofs_s = lofsi[0]
            start = pl.multiple_of((lofs_s // _L) * _L, _L)
            lsel = lofsi % _L
            bx1 = _shuf(x1_v[pl.ds(start, _L)], lsel)
            by1 = _shuf(y1_v[pl.ds(start, _L)], lsel)
            bx2 = _shuf(x2_v[pl.ds(start, _L)], lsel)
            by2 = _shuf(y2_v[pl.ds(start, _L)], lsel)
            pv = jnp.where(lanef == 0.0, mm, lwinf)
            pv = jnp.where(lanef == 2.0, bx1, pv)
            pv = jnp.where(lanef == 3.0, by1, pv)
            pv = jnp.where(lanef == 4.0, bx2, pv)
            pv = jnp.where(lanef == 5.0, by2, pv)
            publoc_v[...] = pv

        m0 = jnp.broadcast_to(jnp.float32(_NEG), (_L,))
        mif0 = jnp.broadcast_to(jnp.float32(0.0), (_L,))
        for i in range(nchunk):
            v = sco_v[pl.ds(i * _L, _L)]
            upd = v > m0
            m0 = jnp.where(upd, v, m0)
            mif0 = jnp.where(upd, jnp.float32(i), mif0)
        publish(m0, mif0)

        def step(t, carry):
            plsc.subcore_barrier()
            pltpu.sync_copy(publoc_v, pub_sh.at[pl.ds(sid * _L, _L)])
            plsc.subcore_barrier()
            pltpu.sync_copy(pub_sh, pubbuf_v)
            c0 = _csplat(0)
            c1 = _csplat(1)
            gm = jnp.broadcast_to(jnp.float32(_NEG), (_L,))
            for tt in range(_NT):
                gm = jnp.maximum(gm, _shuf(pubbuf_v[pl.ds(tt * _L, _L)], c0))
            gif = bigf * jnp.broadcast_to(jnp.float32(1.0), (_L,))
            for tt in range(_NT):
                r = pubbuf_v[pl.ds(tt * _L, _L)]
                bm = _shuf(r, c0)
                bif = _shuf(r, c1)
                gif = jnp.minimum(gif, jnp.where(bm == gm, bif, bigf))
            c2 = _csplat(2)
            c3 = _csplat(3)
            c4 = _csplat(4)
            c5 = _csplat(5)
            zz = jnp.broadcast_to(jnp.float32(0.0), (_L,))
            gx1 = zz
            gy1 = zz
            gx2 = zz
            gy2 = zz
            for tt in range(_NT):
                r = pubbuf_v[pl.ds(tt * _L, _L)]
                hit = _shuf(r, c1) == gif
                gx1 = jnp.where(hit, _shuf(r, c2), gx1)
                gy1 = jnp.where(hit, _shuf(r, c3), gy1)
                gx2 = jnp.where(hit, _shuf(r, c4), gx2)
                gy2 = jnp.where(hit, _shuf(r, c5), gy2)
            barea = (jnp.maximum(gx2 - gx1, 0.0) *
                     jnp.maximum(gy2 - gy1, 0.0))
            m = jnp.broadcast_to(jnp.float32(_NEG), (_L,))
            mif = jnp.broadcast_to(jnp.float32(0.0), (_L,))
            for i in range(nchunk):
                ds = pl.ds(i * _L, _L)
                xx1 = jnp.maximum(gx1, x1_v[ds])
                yy1 = jnp.maximum(gy1, y1_v[ds])
                xx2 = jnp.minimum(gx2, x2_v[ds])
                yy2 = jnp.minimum(gy2, y2_v[ds])
                inter = (jnp.maximum(xx2 - xx1, 0.0) *
                         jnp.maximum(yy2 - yy1, 0.0))
                iou = inter / (barea + ar_v[ds] - inter + 1e-9)
                gidxcf = (jnp.broadcast_to(jnp.float32(base + i * _L), (_L,))
                          + lanef)
                sc = jnp.where(iou > thr, _NEG, sco_v[ds])
                sc = jnp.where(gidxcf == gif, _NEG, sc)
                sco_v[ds] = sc
                upd = sc > m
                m = jnp.where(upd, sc, m)
                mif = jnp.where(upd, jnp.float32(i), mif)
            publish(m, mif)

            @pl.when(sid == 0)
            def _out():
                validf = jnp.where(gm > _NEG / 2,
                                   jnp.broadcast_to(jnp.float32(1.0), (_L,)),
                                   jnp.broadcast_to(jnp.float32(0.0), (_L,)))
                row = jnp.where(lanef == 0.0, gx1, 0.0)
                row = jnp.where(lanef == 1.0, gy1, row)
                row = jnp.where(lanef == 2.0, gx2, row)
                row = jnp.where(lanef == 3.0, gy2, row)
                outbuf_v[pl.ds(t * _L, _L)] = row * validf
            return carry

        lax.fori_loop(0, k, step, jnp.int32(0))

        @pl.when(sid == 0)
        def _flush():
            pltpu.sync_copy(outbuf_v, out_h)


def _pad1d(flat, np_, fill):
    return jnp.pad(flat, (0, np_ - flat.shape[0]), constant_values=fill)


def kernel(rpn_class, rpn_bbox, anchors, image):
    n = rpn_bbox.shape[0]
    k = 1000
    thr = 0.7
    per_tile = ((n + _NT * _L - 1) // (_NT * _L)) * _L
    np_ = per_tile * _NT

    scores = _pad1d(jnp.reshape(rpn_class, (-1,)), np_, _NEG)
    cols = [_pad1d(rpn_bbox[:, i], np_, 0.0) for i in range(4)]
    acols = [_pad1d(anchors[:, i], np_, 0.0) for i in range(4)]

    mesh = plsc.VectorSubcoreMesh(core_axis_name="c", subcore_axis_name="s")
    body = functools.partial(
        _sc_body, k=k, thr=thr,
        hmax=float(image.shape[0] - 1), wmax=float(image.shape[1] - 1),
        per_tile=per_tile)
    f = pl.kernel(
        body,
        mesh=mesh,
        out_type=jax.ShapeDtypeStruct((k * _L,), jnp.float32),
        scratch_types=[pltpu.VMEM((per_tile,), jnp.float32)] * 10
        + [pltpu.VMEM((_L,), jnp.float32),
           pltpu.VMEM((_NT * _L,), jnp.float32),
           pltpu.VMEM((_L,), jnp.int32),
           pltpu.VMEM((k * _L,), jnp.float32),
           pltpu.VMEM_SHARED((_NT * _L,), jnp.float32)],
    )
    out = f(scores, *cols, *acols)
    return out.reshape(k, _L)[:, :4]
